# Initial kernel scaffold; baseline (speedup 1.0000x reference)
import jax
import jax.numpy as jnp
from jax.experimental import pallas as pl


def kernel(feat0, feat1, feat2, feat3, mask_token0, mask_token1, mask_token2, mask_token3, nbr0_src, nbr0_dst, nbr1_src, nbr1_dst, nbr2_src, nbr2_dst, nbr3_src, nbr3_dst, inc_01_edge, inc_01_node, inc_12_bend, inc_12_edge, inc_23_torsion, inc_23_bend):
    raise NotImplementedError("write your pallas kernel here")


# speedup vs baseline: 2.6775x; 2.6775x over previous
import functools

import jax
import jax.numpy as jnp
import numpy as np
from jax.experimental import pallas as pl

_MASK_RATIO = 0.15
_N_R = [50000, 100000, 200000, 300000]
_BLOCK_ROWS = [1000, 1000, 2000, 2000]


@functools.cache
def _masks():
    out = []
    with jax.ensure_compile_time_eval():
        key = jax.random.key(0)
        for r, n in enumerate(_N_R):
            n_mask = max(1, int(n * _MASK_RATIO))
            perm = jax.random.permutation(jax.random.fold_in(key, r), n)[:n_mask]
            m = np.zeros((n,), dtype=np.bool_)
            m[np.asarray(perm)] = True
            out.append(m)
    return out


def _select_kernel(m_ref, t_ref, f_ref, o_ref):
    o_ref[...] = jnp.where(m_ref[...] > 0, t_ref[...], f_ref[...])


def _mask_rows(feat, token, mask_f32, block_rows):
    n, d = feat.shape
    grid = (n // block_rows,)
    return pl.pallas_call(
        _select_kernel,
        grid=grid,
        in_specs=[
            pl.BlockSpec((block_rows, 1), lambda i: (i, 0)),
            pl.BlockSpec((1, d), lambda i: (0, 0)),
            pl.BlockSpec((block_rows, d), lambda i: (i, 0)),
        ],
        out_specs=pl.BlockSpec((block_rows, d), lambda i: (i, 0)),
        out_shape=jax.ShapeDtypeStruct((n, d), feat.dtype),
    )(mask_f32, token.reshape(1, d), feat)


def kernel(feat0, feat1, feat2, feat3, mask_token0, mask_token1, mask_token2, mask_token3, nbr0_src, nbr0_dst, nbr1_src, nbr1_dst, nbr2_src, nbr2_dst, nbr3_src, nbr3_dst, inc_01_edge, inc_01_node, inc_12_bend, inc_12_edge, inc_23_torsion, inc_23_bend):
    feats = [feat0, feat1, feat2, feat3]
    tokens = [mask_token0, mask_token1, mask_token2, mask_token3]
    masks_np = _masks()

    masked_feats = []
    masks = []
    for r, (feat, tok) in enumerate(zip(feats, tokens)):
        m_np = masks_np[r]
        mask_f32 = jnp.asarray(m_np.astype(np.float32)[:, None])
        masked_feats.append(
            _mask_rows(feat, tok.astype(feat.dtype), mask_f32, _BLOCK_ROWS[r])
        )
        masks.append(jnp.asarray(m_np))

    masked_topo = (nbr0_src, nbr0_dst, nbr1_src, nbr1_dst, nbr2_src, nbr2_dst, nbr3_src, nbr3_dst, inc_01_edge, inc_01_node, inc_12_bend, inc_12_edge, inc_23_torsion, inc_23_bend)
    return (*masked_feats, *masks, *masked_topo)


# speedup vs baseline: 2.8198x; 1.0532x over previous
import functools

import jax
import jax.numpy as jnp
import numpy as np
from jax.experimental import pallas as pl

_MASK_RATIO = 0.15
_N_R = [50000, 100000, 200000, 300000]
_D_R = [64, 64, 32, 32]
_PACK = [128 // d for d in _D_R]
_NP_R = [n // p for n, p in zip(_N_R, _PACK)]
_BLK = [1000, 2000, 2000, 3000]
_NBLK = [n // b for n, b in zip(_NP_R, _BLK)]
_SEG_START = [0, 25, 50, 75]
_GRID = 100
_TOPO_E = [800000] * 8 + [200000, 200000, 400000, 400000, 600000, 600000]
_TOPO_3D = {800000: (100, 8, 1000), 200000: (100, 4, 500),
            400000: (100, 8, 500), 600000: (100, 6, 1000)}


def _tf2x32_raw(k1, k2, x0, x1):
    rot0 = (13, 15, 26, 6)
    rot1 = (17, 29, 16, 24)
    ks0 = np.uint32(k1)
    ks1 = np.uint32(k2)
    ks2 = ks0 ^ ks1 ^ np.uint32(0x1BD11BDA)
    x0 = x0.astype(np.uint32) + ks0
    x1 = x1.astype(np.uint32) + ks1

    def rounds(a, b, rots):
        for r in rots:
            a = a + b
            b = (b << np.uint32(r)) | (b >> np.uint32(32 - r))
            b = a ^ b
        return a, b

    for rots, ka, kb, c in ((rot0, ks1, ks2, 1), (rot1, ks2, ks0, 2),
                            (rot0, ks0, ks1, 3), (rot1, ks1, ks2, 4),
                            (rot0, ks2, ks0, 5)):
        x0, x1 = rounds(x0, x1, rots)
        x0 = x0 + ka
        x1 = x1 + kb + np.uint32(c)
    return x0, x1


def _fold_in(key, data):
    x0, x1 = _tf2x32_raw(key[0], key[1],
                         np.zeros(1, np.uint32), np.full(1, data, np.uint32))
    return np.array([x0[0], x1[0]], np.uint32)


def _split2(key):
    b1, b2 = _tf2x32_raw(key[0], key[1],
                         np.zeros(2, np.uint32), np.arange(2, dtype=np.uint32))
    return (np.array([b1[0], b2[0]], np.uint32),
            np.array([b1[1], b2[1]], np.uint32))


def _np_permutation(key, n):
    num_rounds = int(np.ceil(3 * np.log(max(1, n)) / np.log(np.iinfo(np.uint32).max)))
    x = np.arange(n, dtype=np.int32)
    for _ in range(num_rounds):
        key, subkey = _split2(key)
        b1, b2 = _tf2x32_raw(subkey[0], subkey[1],
                             np.zeros(n, np.uint32), np.arange(n, dtype=np.uint32))
        x = x[np.argsort(b1 ^ b2, kind="stable")]
    return x


@functools.cache
def _masks():
    key = np.array([0, 0], np.uint32)
    out = []
    for r, n in enumerate(_N_R):
        n_mask = max(1, int(n * _MASK_RATIO))
        perm = _np_permutation(_fold_in(key, r), n)[:n_mask]
        m = np.zeros((n,), dtype=np.bool_)
        m[perm] = True
        out.append(m)
    return out


def _fused_kernel(*refs):
    m = refs[0:4]
    t = refs[4:8]
    f = refs[8:12]
    tin = refs[12:26]
    fo = refs[26:30]
    tout = refs[30:44]
    i = pl.program_id(0)
    for r in range(4):
        lo = _SEG_START[r]
        hi = lo + _NBLK[r]

        @pl.when((i >= lo) & (i < hi))
        def _(r=r):
            b = _BLK[r]
            d = _D_R[r]
            lane = jax.lax.broadcasted_iota(jnp.int32, (b, 128), 1)
            mm = m[r]
            big = mm[:, _PACK[r] - 1:_PACK[r]]
            for g in range(_PACK[r] - 2, -1, -1):
                big = jnp.where(lane < (g + 1) * d, mm[:, g:g + 1], big)
            fo[r][...] = jnp.where(big > 0, t[r][...], f[r][...])

    for k in range(14):
        tout[k][...] = tin[k][...]


def _feat_map(r):
    lo = _SEG_START[r]
    n = _NBLK[r]
    return lambda i: (jnp.clip(i - lo, 0, n - 1), 0)


def _topo_spec(e):
    _, sub, lanes = _TOPO_3D[e]
    return pl.BlockSpec((1, sub, lanes), lambda i: (i, 0, 0))


@functools.cache
def _fused_call():
    in_specs = (
        [pl.BlockSpec((_BLK[r], _PACK[r]), _feat_map(r)) for r in range(4)]
        + [pl.BlockSpec((1, 128), lambda i: (0, 0)) for _ in range(4)]
        + [pl.BlockSpec((_BLK[r], 128), _feat_map(r)) for r in range(4)]
        + [_topo_spec(e) for e in _TOPO_E]
    )
    out_specs = (
        [pl.BlockSpec((_BLK[r], 128), _feat_map(r)) for r in range(4)]
        + [_topo_spec(e) for e in _TOPO_E]
    )
    out_shape = (
        [jax.ShapeDtypeStruct((n, 128), jnp.float32) for n in _NP_R]
        + [jax.ShapeDtypeStruct(_TOPO_3D[e], jnp.int32) for e in _TOPO_E]
    )
    return pl.pallas_call(
        _fused_kernel,
        grid=(_GRID,),
        in_specs=in_specs,
        out_specs=out_specs,
        out_shape=out_shape,
    )


def kernel(feat0, feat1, feat2, feat3, mask_token0, mask_token1, mask_token2, mask_token3, nbr0_src, nbr0_dst, nbr1_src, nbr1_dst, nbr2_src, nbr2_dst, nbr3_src, nbr3_dst, inc_01_edge, inc_01_node, inc_12_bend, inc_12_edge, inc_23_torsion, inc_23_bend):
    feats = [feat0, feat1, feat2, feat3]
    tokens = [mask_token0, mask_token1, mask_token2, mask_token3]
    topo = [nbr0_src, nbr0_dst, nbr1_src, nbr1_dst, nbr2_src, nbr2_dst, nbr3_src, nbr3_dst, inc_01_edge, inc_01_node, inc_12_bend, inc_12_edge, inc_23_torsion, inc_23_bend]
    masks_np = _masks()

    mask_f32 = [jnp.asarray(m.astype(np.float32).reshape(-1, p))
                for m, p in zip(masks_np, _PACK)]
    tok128 = [jnp.tile(t.astype(jnp.float32), p).reshape(1, 128)
              for t, p in zip(tokens, _PACK)]
    feats_pk = [f.reshape(-1, 128) for f in feats]
    topo3d = [a.reshape(_TOPO_3D[a.shape[0]]) for a in topo]

    outs = _fused_call()(*mask_f32, *tok128, *feats_pk, *topo3d)
    masked_feats = [o.reshape(n, d) for o, n, d in zip(outs[0:4], _N_R, _D_R)]
    masked_topo = [o.reshape(-1) for o in outs[4:18]]
    masks = [jnp.asarray(m) for m in masks_np]
    return (*masked_feats, *masks, *masked_topo)


# speedup vs baseline: 3.0297x; 1.0744x over previous
import functools

import jax
import jax.numpy as jnp
import numpy as np
from jax.experimental import pallas as pl

_MASK_RATIO = 0.15
_N_R = [50000, 100000, 200000, 300000]
_D_R = [64, 64, 32, 32]
_PACK = [128 // d for d in _D_R]
_NP_R = [n // p for n, p in zip(_N_R, _PACK)]
_GRID = 25
_BLK = [n // _GRID for n in _NP_R]
_TOPO_E = [800000] * 8 + [200000, 200000, 400000, 400000, 600000, 600000]
_TOPO_3D = {800000: (25, 8, 4000), 200000: (25, 8, 1000),
            400000: (25, 8, 2000), 600000: (25, 8, 3000)}


def _tf2x32_raw(k1, k2, x0, x1):
    rot0 = (13, 15, 26, 6)
    rot1 = (17, 29, 16, 24)
    ks0 = np.uint32(k1)
    ks1 = np.uint32(k2)
    ks2 = ks0 ^ ks1 ^ np.uint32(0x1BD11BDA)
    x0 = x0.astype(np.uint32) + ks0
    x1 = x1.astype(np.uint32) + ks1

    def rounds(a, b, rots):
        for r in rots:
            a = a + b
            b = (b << np.uint32(r)) | (b >> np.uint32(32 - r))
            b = a ^ b
        return a, b

    for rots, ka, kb, c in ((rot0, ks1, ks2, 1), (rot1, ks2, ks0, 2),
                            (rot0, ks0, ks1, 3), (rot1, ks1, ks2, 4),
                            (rot0, ks2, ks0, 5)):
        x0, x1 = rounds(x0, x1, rots)
        x0 = x0 + ka
        x1 = x1 + kb + np.uint32(c)
    return x0, x1


def _fold_in(key, data):
    x0, x1 = _tf2x32_raw(key[0], key[1],
                         np.zeros(1, np.uint32), np.full(1, data, np.uint32))
    return np.array([x0[0], x1[0]], np.uint32)


def _split2(key):
    b1, b2 = _tf2x32_raw(key[0], key[1],
                         np.zeros(2, np.uint32), np.arange(2, dtype=np.uint32))
    return (np.array([b1[0], b2[0]], np.uint32),
            np.array([b1[1], b2[1]], np.uint32))


def _np_permutation(key, n):
    num_rounds = int(np.ceil(3 * np.log(max(1, n)) / np.log(np.iinfo(np.uint32).max)))
    x = np.arange(n, dtype=np.int32)
    for _ in range(num_rounds):
        key, subkey = _split2(key)
        b1, b2 = _tf2x32_raw(subkey[0], subkey[1],
                             np.zeros(n, np.uint32), np.arange(n, dtype=np.uint32))
        x = x[np.argsort(b1 ^ b2, kind="stable")]
    return x


@functools.cache
def _masks():
    key = np.array([0, 0], np.uint32)
    out = []
    for r, n in enumerate(_N_R):
        n_mask = max(1, int(n * _MASK_RATIO))
        perm = _np_permutation(_fold_in(key, r), n)[:n_mask]
        m = np.zeros((n,), dtype=np.bool_)
        m[perm] = True
        out.append(m)
    return out


def _fused_kernel(*refs):
    m = refs[0:4]
    t = refs[4:8]
    f = refs[8:12]
    tin = refs[12:26]
    fo = refs[26:30]
    tout = refs[30:44]
    for r in range(4):
        b = _BLK[r]
        d = _D_R[r]
        lane = jax.lax.broadcasted_iota(jnp.int32, (b, 128), 1)
        mm = m[r]
        big = mm[:, _PACK[r] - 1:_PACK[r]]
        for g in range(_PACK[r] - 2, -1, -1):
            big = jnp.where(lane < (g + 1) * d, mm[:, g:g + 1], big)
        fo[r][...] = jnp.where(big > 0, t[r][...], f[r][...])

    for k in range(14):
        tout[k][...] = tin[k][...]


def _feat_map(r):
    return lambda i: (i, 0)


def _topo_spec(e):
    _, sub, lanes = _TOPO_3D[e]
    return pl.BlockSpec((1, sub, lanes), lambda i: (i, 0, 0))


@functools.cache
def _fused_call():
    in_specs = (
        [pl.BlockSpec((_BLK[r], _PACK[r]), _feat_map(r)) for r in range(4)]
        + [pl.BlockSpec((1, 128), lambda i: (0, 0)) for _ in range(4)]
        + [pl.BlockSpec((_BLK[r], 128), _feat_map(r)) for r in range(4)]
        + [_topo_spec(e) for e in _TOPO_E]
    )
    out_specs = (
        [pl.BlockSpec((_BLK[r], 128), _feat_map(r)) for r in range(4)]
        + [_topo_spec(e) for e in _TOPO_E]
    )
    out_shape = (
        [jax.ShapeDtypeStruct((n, 128), jnp.float32) for n in _NP_R]
        + [jax.ShapeDtypeStruct(_TOPO_3D[e], jnp.int32) for e in _TOPO_E]
    )
    return pl.pallas_call(
        _fused_kernel,
        grid=(_GRID,),
        in_specs=in_specs,
        out_specs=out_specs,
        out_shape=out_shape,
    )


def kernel(feat0, feat1, feat2, feat3, mask_token0, mask_token1, mask_token2, mask_token3, nbr0_src, nbr0_dst, nbr1_src, nbr1_dst, nbr2_src, nbr2_dst, nbr3_src, nbr3_dst, inc_01_edge, inc_01_node, inc_12_bend, inc_12_edge, inc_23_torsion, inc_23_bend):
    feats = [feat0, feat1, feat2, feat3]
    tokens = [mask_token0, mask_token1, mask_token2, mask_token3]
    topo = [nbr0_src, nbr0_dst, nbr1_src, nbr1_dst, nbr2_src, nbr2_dst, nbr3_src, nbr3_dst, inc_01_edge, inc_01_node, inc_12_bend, inc_12_edge, inc_23_torsion, inc_23_bend]
    masks_np = _masks()

    mask_f32 = [jnp.asarray(m.astype(np.float32).reshape(-1, p))
                for m, p in zip(masks_np, _PACK)]
    tok128 = [jnp.tile(t.astype(jnp.float32), p).reshape(1, 128)
              for t, p in zip(tokens, _PACK)]
    feats_pk = [f.reshape(-1, 128) for f in feats]
    topo3d = [a.reshape(_TOPO_3D[a.shape[0]]) for a in topo]

    outs = _fused_call()(*mask_f32, *tok128, *feats_pk, *topo3d)
    masked_feats = [o.reshape(n, d) for o, n, d in zip(outs[0:4], _N_R, _D_R)]
    masked_topo = [o.reshape(-1) for o in outs[4:18]]
    masks = [jnp.asarray(m) for m in masks_np]
    return (*masked_feats, *masks, *masked_topo)


# speedup vs baseline: 22.3137x; 7.3650x over previous
import functools

import jax
import jax.numpy as jnp
import numpy as np
from jax.experimental import pallas as pl

_MASK_RATIO = 0.15
_N_R = [50000, 100000, 200000, 300000]
_D_R = [64, 64, 32, 32]
_GRID = 25
_CLN = [2048, 4096, 8192, 12032]


def _tf2x32_raw(k1, k2, x0, x1):
    rot0 = (13, 15, 26, 6)
    rot1 = (17, 29, 16, 24)
    ks0 = np.uint32(k1)
    ks1 = np.uint32(k2)
    ks2 = ks0 ^ ks1 ^ np.uint32(0x1BD11BDA)
    x0 = x0.astype(np.uint32) + ks0
    x1 = x1.astype(np.uint32) + ks1

    def rounds(a, b, rots):
        for r in rots:
            a = a + b
            b = (b << np.uint32(r)) | (b >> np.uint32(32 - r))
            b = a ^ b
        return a, b

    for rots, ka, kb, c in ((rot0, ks1, ks2, 1), (rot1, ks2, ks0, 2),
                            (rot0, ks0, ks1, 3), (rot1, ks1, ks2, 4),
                            (rot0, ks2, ks0, 5)):
        x0, x1 = rounds(x0, x1, rots)
        x0 = x0 + ka
        x1 = x1 + kb + np.uint32(c)
    return x0, x1


def _fold_in(key, data):
    x0, x1 = _tf2x32_raw(key[0], key[1],
                         np.zeros(1, np.uint32), np.full(1, data, np.uint32))
    return np.array([x0[0], x1[0]], np.uint32)


def _split2(key):
    b1, b2 = _tf2x32_raw(key[0], key[1],
                         np.zeros(2, np.uint32), np.arange(2, dtype=np.uint32))
    return (np.array([b1[0], b2[0]], np.uint32),
            np.array([b1[1], b2[1]], np.uint32))


def _np_permutation(key, n):
    num_rounds = int(np.ceil(3 * np.log(max(1, n)) / np.log(np.iinfo(np.uint32).max)))
    x = np.arange(n, dtype=np.int32)
    for _ in range(num_rounds):
        key, subkey = _split2(key)
        b1, b2 = _tf2x32_raw(subkey[0], subkey[1],
                             np.zeros(n, np.uint32), np.arange(n, dtype=np.uint32))
        x = x[np.argsort(b1 ^ b2, kind="stable")]
    return x


@functools.cache
def _masks():
    key = np.array([0, 0], np.uint32)
    out = []
    for r, n in enumerate(_N_R):
        n_mask = max(1, int(n * _MASK_RATIO))
        perm = _np_permutation(_fold_in(key, r), n)[:n_mask]
        m = np.zeros((n,), dtype=np.bool_)
        m[perm] = True
        out.append(m)
    return out


def _fused_kernel(*refs):
    m = refs[0:4]
    t = refs[4:8]
    f = refs[8:12]
    o = refs[12:16]
    for r in range(4):
        o[r][...] = jnp.where(m[r][...] > 0, t[r][...], f[r][...])


@functools.cache
def _fused_call():
    in_specs = (
        [pl.BlockSpec((1, _CLN[r]), lambda i, r=r: (0, i)) for r in range(4)]
        + [pl.BlockSpec((_D_R[r], 1), lambda i: (0, 0)) for r in range(4)]
        + [pl.BlockSpec((_D_R[r], _CLN[r]), lambda i, r=r: (0, i))
           for r in range(4)]
    )
    out_specs = [pl.BlockSpec((_D_R[r], _CLN[r]), lambda i, r=r: (0, i))
                 for r in range(4)]
    out_shape = [jax.ShapeDtypeStruct((d, n), jnp.float32)
                 for n, d in zip(_N_R, _D_R)]
    return pl.pallas_call(
        _fused_kernel,
        grid=(_GRID,),
        in_specs=in_specs,
        out_specs=out_specs,
        out_shape=out_shape,
    )


def kernel(feat0, feat1, feat2, feat3, mask_token0, mask_token1, mask_token2, mask_token3, nbr0_src, nbr0_dst, nbr1_src, nbr1_dst, nbr2_src, nbr2_dst, nbr3_src, nbr3_dst, inc_01_edge, inc_01_node, inc_12_bend, inc_12_edge, inc_23_torsion, inc_23_bend):
    feats = [feat0, feat1, feat2, feat3]
    tokens = [mask_token0, mask_token1, mask_token2, mask_token3]
    masks_np = _masks()

    mask_f32 = [jnp.asarray(m.astype(np.float32).reshape(1, -1))
                for m in masks_np]
    tok2d = [t.astype(jnp.float32).reshape(-1, 1) for t in tokens]
    feats_t = [f.T for f in feats]

    outs = _fused_call()(*mask_f32, *tok2d, *feats_t)
    masked_feats = [o.T for o in outs]
    masked_topo = (nbr0_src, nbr0_dst, nbr1_src, nbr1_dst, nbr2_src, nbr2_dst, nbr3_src, nbr3_dst, inc_01_edge, inc_01_node, inc_12_bend, inc_12_edge, inc_23_torsion, inc_23_bend)
    masks = [jnp.asarray(m) for m in masks_np]
    return (*masked_feats, *masks, *masked_topo)


# speedup vs baseline: 22.5584x; 1.0110x over previous
import functools

import jax
import jax.numpy as jnp
import numpy as np
from jax.experimental import pallas as pl

_MASK_RATIO = 0.15
_N_R = [50000, 100000, 200000, 300000]
_D_R = [64, 64, 32, 32]
_GRID = 13
_CLN = [3968, 7808, 15488, 23168]


def _tf2x32_raw(k1, k2, x0, x1):
    rot0 = (13, 15, 26, 6)
    rot1 = (17, 29, 16, 24)
    ks0 = np.uint32(k1)
    ks1 = np.uint32(k2)
    ks2 = ks0 ^ ks1 ^ np.uint32(0x1BD11BDA)
    x0 = x0.astype(np.uint32) + ks0
    x1 = x1.astype(np.uint32) + ks1

    def rounds(a, b, rots):
        for r in rots:
            a = a + b
            b = (b << np.uint32(r)) | (b >> np.uint32(32 - r))
            b = a ^ b
        return a, b

    for rots, ka, kb, c in ((rot0, ks1, ks2, 1), (rot1, ks2, ks0, 2),
                            (rot0, ks0, ks1, 3), (rot1, ks1, ks2, 4),
                            (rot0, ks2, ks0, 5)):
        x0, x1 = rounds(x0, x1, rots)
        x0 = x0 + ka
        x1 = x1 + kb + np.uint32(c)
    return x0, x1


def _fold_in(key, data):
    x0, x1 = _tf2x32_raw(key[0], key[1],
                         np.zeros(1, np.uint32), np.full(1, data, np.uint32))
    return np.array([x0[0], x1[0]], np.uint32)


def _split2(key):
    b1, b2 = _tf2x32_raw(key[0], key[1],
                         np.zeros(2, np.uint32), np.arange(2, dtype=np.uint32))
    return (np.array([b1[0], b2[0]], np.uint32),
            np.array([b1[1], b2[1]], np.uint32))


def _np_permutation(key, n):
    num_rounds = int(np.ceil(3 * np.log(max(1, n)) / np.log(np.iinfo(np.uint32).max)))
    x = np.arange(n, dtype=np.int32)
    for _ in range(num_rounds):
        key, subkey = _split2(key)
        b1, b2 = _tf2x32_raw(subkey[0], subkey[1],
                             np.zeros(n, np.uint32), np.arange(n, dtype=np.uint32))
        x = x[np.argsort(b1 ^ b2, kind="stable")]
    return x


@functools.cache
def _masks():
    key = np.array([0, 0], np.uint32)
    out = []
    for r, n in enumerate(_N_R):
        n_mask = max(1, int(n * _MASK_RATIO))
        perm = _np_permutation(_fold_in(key, r), n)[:n_mask]
        m = np.zeros((n,), dtype=np.bool_)
        m[perm] = True
        out.append(m)
    return out


def _fused_kernel(*refs):
    m = refs[0:4]
    t = refs[4:8]
    f = refs[8:12]
    o = refs[12:16]
    for r in range(4):
        o[r][...] = jnp.where(m[r][...] > 0, t[r][...], f[r][...])


@functools.cache
def _fused_call():
    in_specs = (
        [pl.BlockSpec((1, _CLN[r]), lambda i, r=r: (0, i)) for r in range(4)]
        + [pl.BlockSpec((_D_R[r], 1), lambda i: (0, 0)) for r in range(4)]
        + [pl.BlockSpec((_D_R[r], _CLN[r]), lambda i, r=r: (0, i))
           for r in range(4)]
    )
    out_specs = [pl.BlockSpec((_D_R[r], _CLN[r]), lambda i, r=r: (0, i))
                 for r in range(4)]
    out_shape = [jax.ShapeDtypeStruct((d, n), jnp.float32)
                 for n, d in zip(_N_R, _D_R)]
    return pl.pallas_call(
        _fused_kernel,
        grid=(_GRID,),
        in_specs=in_specs,
        out_specs=out_specs,
        out_shape=out_shape,
    )


def kernel(feat0, feat1, feat2, feat3, mask_token0, mask_token1, mask_token2, mask_token3, nbr0_src, nbr0_dst, nbr1_src, nbr1_dst, nbr2_src, nbr2_dst, nbr3_src, nbr3_dst, inc_01_edge, inc_01_node, inc_12_bend, inc_12_edge, inc_23_torsion, inc_23_bend):
    feats = [feat0, feat1, feat2, feat3]
    tokens = [mask_token0, mask_token1, mask_token2, mask_token3]
    masks_np = _masks()

    mask_f32 = [jnp.asarray(m.astype(np.float32).reshape(1, -1))
                for m in masks_np]
    tok2d = [t.astype(jnp.float32).reshape(-1, 1) for t in tokens]
    feats_t = [f.T for f in feats]

    outs = _fused_call()(*mask_f32, *tok2d, *feats_t)
    masked_feats = [o.T for o in outs]
    masked_topo = (nbr0_src, nbr0_dst, nbr1_src, nbr1_dst, nbr2_src, nbr2_dst, nbr3_src, nbr3_dst, inc_01_edge, inc_01_node, inc_12_bend, inc_12_edge, inc_23_torsion, inc_23_bend)
    masks = [jnp.asarray(m) for m in masks_np]
    return (*masked_feats, *masks, *masked_topo)
